# baseline (device time: 15819 ns/iter reference)
import jax
import jax.numpy as jnp
from jax import lax
from jax.experimental import pallas as pl
from jax.experimental.pallas import tpu as pltpu

N_DEV = 4
B, SQ, SKV, HQ, DH = 2, 128, 128, 16, 64
H_LOC = HQ // N_DEV
D_MODEL = 512
BF16 = jnp.bfloat16


def kernel(x, Wq, K_ext, V_ext, Wo):
    my = lax.axis_index("i")

    x2d = x.reshape(B * SQ, D_MODEL).astype(BF16)
    wq = Wq.astype(BF16)
    wo = Wo.astype(BF16)
    k_loc = lax.dynamic_slice_in_dim(K_ext, my * H_LOC, H_LOC, axis=2)
    v_loc = lax.dynamic_slice_in_dim(V_ext, my * H_LOC, H_LOC, axis=2)
    k_loc = k_loc.transpose(0, 2, 1, 3).reshape(B * H_LOC, SKV, DH).astype(BF16)
    v_loc = v_loc.transpose(0, 2, 1, 3).reshape(B * H_LOC, SKV, DH).astype(BF16)

    def body(x_ref, wq_ref, k_ref, v_ref, wo_ref, out_ref,
             comm_ref, send_sems, recv_sems):
        my_pos = lax.axis_index("i")
        p1 = my_pos ^ 1
        p2 = my_pos ^ 3

        barrier_sem = pltpu.get_barrier_semaphore()
        for nbr in (p1, p2):
            pl.semaphore_signal(
                barrier_sem, inc=1,
                device_id=(nbr,), device_id_type=pl.DeviceIdType.MESH,
            )
        pl.semaphore_wait(barrier_sem, 2)

        def exchange(src_slot, dst_slot, partner, sem_idx):
            return pltpu.make_async_remote_copy(
                src_ref=comm_ref.at[src_slot],
                dst_ref=comm_ref.at[dst_slot],
                send_sem=send_sems.at[sem_idx],
                recv_sem=recv_sems.at[sem_idx],
                device_id=(partner,),
                device_id_type=pl.DeviceIdType.MESH,
            )

        q2d = jnp.dot(x_ref[...], wq_ref[...],
                      preferred_element_type=jnp.float32).astype(BF16)

        ri = lax.broadcasted_iota(jnp.int32, (SQ, SKV), 0) // 64
        ci = lax.broadcasted_iota(jnp.int32, (SQ, SKV), 1) // 64
        mask = (ri == ci) | ((ci % 4) == (ri % 4))

        rdma = [None, None]
        for b in range(B):
            acc = jnp.zeros((SQ, D_MODEL), jnp.float32)
            for h in range(H_LOC):
                bh = b * H_LOC + h
                q = q2d[b * SQ:(b + 1) * SQ, h * DH:(h + 1) * DH]
                k = k_ref[bh]
                s = lax.dot_general(
                    q, k, (((1,), (1,)), ((), ())),
                    preferred_element_type=jnp.float32) * 0.125
                s = jnp.where(mask, s, -1e9)
                m = jnp.max(s, axis=1, keepdims=True)
                w = jnp.exp(s - m)
                w = w / jnp.sum(w, axis=1, keepdims=True)
                ctx = jnp.dot(w.astype(BF16), v_ref[bh],
                              preferred_element_type=jnp.float32)
                acc = acc + jnp.dot(ctx.astype(BF16),
                                    wo_ref[h * DH:(h + 1) * DH, :],
                                    preferred_element_type=jnp.float32)
            comm_ref[b] = acc.astype(BF16)
            rdma[b] = exchange(b, 2 + b, p1 if b == 0 else p2, b)
            rdma[b].start()

        rdma[0].wait_recv()
        comm_ref[2] = comm_ref[0, :, :] + comm_ref[2, :, :]
        r2a = exchange(2, 4, p2, 2)
        r2a.start()

        rdma[1].wait_recv()
        comm_ref[3] = comm_ref[1, :, :] + comm_ref[3, :, :]
        r2b = exchange(3, 5, p1, 3)
        r2b.start()

        r2a.wait_recv()
        out_ref[0] = (comm_ref[2, :, :].astype(jnp.float32)
                      + comm_ref[4, :, :].astype(jnp.float32))
        r2b.wait_recv()
        out_ref[1] = (comm_ref[3, :, :].astype(jnp.float32)
                      + comm_ref[5, :, :].astype(jnp.float32))

        rdma[0].wait_send()
        rdma[1].wait_send()
        r2a.wait_send()
        r2b.wait_send()

    return pl.pallas_call(
        body,
        out_shape=jax.ShapeDtypeStruct((B, SQ, D_MODEL), jnp.float32),
        in_specs=[pl.BlockSpec(memory_space=pltpu.VMEM)] * 5,
        out_specs=pl.BlockSpec(memory_space=pltpu.VMEM),
        scratch_shapes=[
            pltpu.VMEM((6, SQ, D_MODEL), BF16),
            pltpu.SemaphoreType.DMA((4,)),
            pltpu.SemaphoreType.DMA((4,)),
        ],
        compiler_params=pltpu.CompilerParams(collective_id=0),
    )(x2d, wq, k_loc, v_loc, wo)
